# baseline (speedup 1.0000x reference)
import functools

import jax
import jax.numpy as jnp
from jax import lax
from jax.experimental import pallas as pl
from jax.experimental.pallas import tpu as pltpu
from jax.experimental.pallas import tpu_sc as plsc

EMBED = 64
HIDDEN = 256
GRP = 8
NC = 2
NS = 16
NW = NC * NS
VEC = 16


def _sc_group_gather(X3, gid):
    B = gid.shape[0]
    per_w = B // NW
    mesh = plsc.VectorSubcoreMesh(core_axis_name="c", subcore_axis_name="s")
    out_t = jax.ShapeDtypeStruct((B, GRP, EMBED), jnp.float32)

    @functools.partial(
        pl.kernel,
        mesh=mesh,
        out_type=out_t,
        scratch_types=[
            pltpu.VMEM((per_w,), jnp.int32),
            pltpu.SemaphoreType.DMA,
        ],
    )
    def gather_kernel(x_hbm, g_hbm, o_hbm, g_v, sem):
        wid = lax.axis_index("s") * NC + lax.axis_index("c")
        base = wid * per_w
        pltpu.sync_copy(g_hbm.at[pl.ds(base, per_w)], g_v)

        @pl.loop(0, per_w // VEC)
        def _issue(g):
            vg = g_v[pl.ds(g * VEC, VEC)]
            for k in range(VEC):
                pltpu.async_copy(x_hbm.at[pl.ds(vg[k], 1)],
                                 o_hbm.at[pl.ds(base + g * VEC + k, 1)], sem)

        pltpu.make_async_copy(x_hbm.at[pl.ds(0, per_w)],
                              o_hbm.at[pl.ds(base, per_w)], sem).wait()

    return gather_kernel(X3, gid)


def _mlp_body(xu_ref, xv_ref, us_ref, vs_ref, w1_ref, b1_ref,
              w2_ref, b2_ref, o_ref):
    us = us_ref[...]
    vs = vs_ref[...]
    u = jnp.zeros_like(xu_ref[:, 0, :])
    v = jnp.zeros_like(u)
    for a in range(GRP):
        u = u + jnp.where(us == a, xu_ref[:, a, :], 0.0)
        v = v + jnp.where(vs == a, xv_ref[:, a, :], 0.0)
    x = jnp.concatenate([u, v], axis=1)
    h = jnp.maximum(
        lax.dot_general(x, w1_ref[...], (((1,), (0,)), ((), ())),
                        preferred_element_type=jnp.float32)
        + b1_ref[...], 0.0)
    mlp = jnp.sum(h * w2_ref[...].T, axis=1, keepdims=True)
    dot = jnp.sum(u * v, axis=1, keepdims=True)
    o_ref[...] = dot + mlp + b2_ref[...]


def _tc_mlp(xu, xv, u_sel, v_sel, W1, b1, W2, b2):
    B = xu.shape[0]
    BLK = 2048
    grid = (B // BLK,)
    b1r = b1.reshape(1, HIDDEN)
    b2r = b2.reshape(1, 1)
    return pl.pallas_call(
        _mlp_body,
        grid=grid,
        in_specs=[
            pl.BlockSpec((BLK, GRP, EMBED), lambda i: (i, 0, 0)),
            pl.BlockSpec((BLK, GRP, EMBED), lambda i: (i, 0, 0)),
            pl.BlockSpec((BLK, 1), lambda i: (i, 0)),
            pl.BlockSpec((BLK, 1), lambda i: (i, 0)),
            pl.BlockSpec((2 * EMBED, HIDDEN), lambda i: (0, 0)),
            pl.BlockSpec((1, HIDDEN), lambda i: (0, 0)),
            pl.BlockSpec((HIDDEN, 1), lambda i: (0, 0)),
            pl.BlockSpec((1, 1), lambda i: (0, 0)),
        ],
        out_specs=pl.BlockSpec((BLK, 1), lambda i: (i, 0)),
        out_shape=jax.ShapeDtypeStruct((B, 1), jnp.float32),
    )(xu, xv, u_sel, v_sel, W1, b1r, W2, b2r)


@jax.jit
def kernel(user_ids, item_ids, user_table, item_table, W1, b1, W2, b2):
    B = user_ids.shape[0]
    n_grp = user_table.shape[0] // GRP
    Xu = user_table.reshape(n_grp, GRP, EMBED)
    Xi = item_table.reshape(n_grp, GRP, EMBED)
    gu = _sc_group_gather(Xu, user_ids // GRP)
    gi = _sc_group_gather(Xi, item_ids // GRP)
    u_sel = (user_ids % GRP).reshape(-1, 1)
    v_sel = (item_ids % GRP).reshape(-1, 1)
    del B
    return _tc_mlp(gu, gi, u_sel, v_sel, W1, b1, W2, b2)


# speedup vs baseline: 6.9532x; 6.9532x over previous
import functools

import jax
import jax.numpy as jnp
from jax import lax
from jax.experimental import pallas as pl
from jax.experimental.pallas import tpu as pltpu
from jax.experimental.pallas import tpu_sc as plsc

EMBED = 64
HIDDEN = 256
GRP = 8
NC = 2
NS = 16
NW = NC * NS
VEC = 16


def _sc_group_gather(X3, gid):
    B = gid.shape[0]
    per_w = B // NW
    mesh = plsc.VectorSubcoreMesh(core_axis_name="c", subcore_axis_name="s")
    out_t = jax.ShapeDtypeStruct((B, GRP, EMBED), jnp.float32)

    CHUNK = 32

    @functools.partial(
        pl.kernel,
        mesh=mesh,
        out_type=out_t,
        scratch_types=[
            pltpu.VMEM((per_w,), jnp.int32),
            pltpu.VMEM((CHUNK, GRP, EMBED), jnp.float32),
            pltpu.VMEM((CHUNK, GRP, EMBED), jnp.float32),
            pltpu.SemaphoreType.DMA,
            pltpu.SemaphoreType.DMA,
        ],
    )
    def gather_kernel(x_hbm, g_hbm, o_hbm, g_v, b0_v, b1_v, sem0, sem1):
        wid = lax.axis_index("s") * NC + lax.axis_index("c")
        base = wid * per_w
        pltpu.sync_copy(g_hbm.at[pl.ds(base, per_w)], g_v)

        def issue(c, buf, sem):
            off = c * CHUNK

            @pl.loop(0, CHUNK // VEC)
            def _issue(g):
                vg = g_v[pl.ds(off + g * VEC, VEC)]
                for k in range(VEC):
                    pltpu.async_copy(x_hbm.at[pl.ds(vg[k], 1)],
                                     buf.at[pl.ds(g * VEC + k, 1)], sem)

        def drain_store(c, buf, sem):
            off = c * CHUNK
            pltpu.make_async_copy(x_hbm.at[pl.ds(0, CHUNK)], buf, sem).wait()
            pltpu.sync_copy(buf, o_hbm.at[pl.ds(base + off, CHUNK)])

        issue(0, b0_v, sem0)

        @pl.loop(0, per_w // CHUNK - 1)
        def _steps(c):
            pl.when((c % 2) == 0)(lambda: issue(c + 1, b1_v, sem1))
            pl.when((c % 2) == 1)(lambda: issue(c + 1, b0_v, sem0))
            pl.when((c % 2) == 0)(lambda: drain_store(c, b0_v, sem0))
            pl.when((c % 2) == 1)(lambda: drain_store(c, b1_v, sem1))

        last = per_w // CHUNK - 1
        pl.when((last % 2) == 0)(lambda: drain_store(last, b0_v, sem0))
        pl.when((last % 2) == 1)(lambda: drain_store(last, b1_v, sem1))

    return gather_kernel(X3, gid)


def _mlp_body(xu_ref, xv_ref, us_ref, vs_ref, w1_ref, b1_ref,
              w2_ref, b2_ref, o_ref):
    us = us_ref[...]
    vs = vs_ref[...]
    u = jnp.zeros_like(xu_ref[:, 0, :])
    v = jnp.zeros_like(u)
    for a in range(GRP):
        u = u + jnp.where(us == a, xu_ref[:, a, :], 0.0)
        v = v + jnp.where(vs == a, xv_ref[:, a, :], 0.0)
    x = jnp.concatenate([u, v], axis=1)
    h = jnp.maximum(
        lax.dot_general(x, w1_ref[...], (((1,), (0,)), ((), ())),
                        preferred_element_type=jnp.float32)
        + b1_ref[...], 0.0)
    mlp = jnp.sum(h * w2_ref[...].T, axis=1, keepdims=True)
    dot = jnp.sum(u * v, axis=1, keepdims=True)
    o_ref[...] = dot + mlp + b2_ref[...]


def _tc_mlp(xu, xv, u_sel, v_sel, W1, b1, W2, b2):
    B = xu.shape[0]
    BLK = 2048
    grid = (B // BLK,)
    b1r = b1.reshape(1, HIDDEN)
    b2r = b2.reshape(1, 1)
    return pl.pallas_call(
        _mlp_body,
        grid=grid,
        in_specs=[
            pl.BlockSpec((BLK, GRP, EMBED), lambda i: (i, 0, 0)),
            pl.BlockSpec((BLK, GRP, EMBED), lambda i: (i, 0, 0)),
            pl.BlockSpec((BLK, 1), lambda i: (i, 0)),
            pl.BlockSpec((BLK, 1), lambda i: (i, 0)),
            pl.BlockSpec((2 * EMBED, HIDDEN), lambda i: (0, 0)),
            pl.BlockSpec((1, HIDDEN), lambda i: (0, 0)),
            pl.BlockSpec((HIDDEN, 1), lambda i: (0, 0)),
            pl.BlockSpec((1, 1), lambda i: (0, 0)),
        ],
        out_specs=pl.BlockSpec((BLK, 1), lambda i: (i, 0)),
        out_shape=jax.ShapeDtypeStruct((B, 1), jnp.float32),
    )(xu, xv, u_sel, v_sel, W1, b1r, W2, b2r)


@jax.jit
def kernel(user_ids, item_ids, user_table, item_table, W1, b1, W2, b2):
    B = user_ids.shape[0]
    n_grp = user_table.shape[0] // GRP
    Xu = user_table.reshape(n_grp, GRP, EMBED)
    Xi = item_table.reshape(n_grp, GRP, EMBED)
    gu = _sc_group_gather(Xu, user_ids // GRP)
    gi = _sc_group_gather(Xi, item_ids // GRP)
    u_sel = (user_ids % GRP).reshape(-1, 1)
    v_sel = (item_ids % GRP).reshape(-1, 1)
    del B
    return _tc_mlp(gu, gi, u_sel, v_sel, W1, b1, W2, b2)


# speedup vs baseline: 8.6740x; 1.2475x over previous
import dataclasses
import functools

import jax
import jax.numpy as jnp
from jax import lax
from jax.experimental import pallas as pl
from jax.experimental.pallas import tpu as pltpu
from jax.experimental.pallas import tpu_sc as plsc

EMBED = 64
HIDDEN = 256
NC = 2
NS = 16
NW = NC * NS
VEC = 16
NBUF = 4
SLAB = 128
NROWS = 1000000
TAIL = NROWS - (NROWS % SLAB)


def _sc_native_gather(Tu, Ti, Tu_tail, Ti_tail, uids, iids, iota16):
    B = uids.shape[0]
    per_w = B // NW
    mesh = plsc.VectorSubcoreMesh(core_axis_name="c", subcore_axis_name="s")
    out_t = jax.ShapeDtypeStruct((B, EMBED), jnp.float32)

    cp = pltpu.CompilerParams()
    if "needs_layout_passes" in pltpu.CompilerParams.__dataclass_fields__:
        cp = dataclasses.replace(cp, needs_layout_passes=False)

    @functools.partial(
        pl.kernel,
        mesh=mesh,
        out_type=(out_t, out_t),
        compiler_params=cp,
        scratch_types=[
            pltpu.VMEM((per_w,), jnp.int32),
            pltpu.VMEM((per_w,), jnp.int32),
            pltpu.VMEM((VEC,), jnp.int32),
            pltpu.VMEM((NBUF, EMBED, SLAB), jnp.float32),
            pltpu.VMEM((EMBED, NROWS - TAIL), jnp.float32),
            pltpu.VMEM((EMBED, NROWS - TAIL), jnp.float32),
            pltpu.VMEM((per_w, EMBED), jnp.float32),
            pltpu.SemaphoreType.DMA,
            pltpu.SemaphoreType.DMA,
            pltpu.SemaphoreType.DMA,
            pltpu.SemaphoreType.DMA,
        ],
    )
    def gather_kernel(tu_hbm, ti_hbm, tut_hbm, tit_hbm, ui_hbm, ii_hbm,
                      io_hbm, ou_hbm, oi_hbm,
                      ui_v, ii_v, io_v, slab_v, tu_tail_v, ti_tail_v,
                      stage_v, s0, s1, s2, s3):
        sems = (s0, s1, s2, s3)
        wid = lax.axis_index("s") * NC + lax.axis_index("c")
        base = wid * per_w
        pltpu.sync_copy(ui_hbm.at[pl.ds(base, per_w)], ui_v)
        pltpu.sync_copy(ii_hbm.at[pl.ds(base, per_w)], ii_v)
        pltpu.sync_copy(io_hbm, io_v)
        pltpu.sync_copy(tut_hbm, tu_tail_v)
        pltpu.sync_copy(tit_hbm, ti_tail_v)
        rows0 = io_v[pl.ds(0, VEC)]

        def fetch(t_hbm, idv, slot):
            sl = (idv >> 7) << 7

            @pl.when(sl < TAIL)
            def _():
                slq = pl.multiple_of(sl, SLAB)
                pltpu.async_copy(t_hbm.at[:, pl.ds(slq, SLAB)],
                                 slab_v.at[slot], sems[slot])

        def extract(t_hbm, tail_v, idv, j, slot):
            sl = (idv >> 7) << 7
            tail = sl >= TAIL

            @pl.when(jnp.logical_not(tail))
            def _():
                pltpu.make_async_copy(
                    t_hbm.at[:, pl.ds(0, SLAB)], slab_v.at[slot],
                    sems[slot]).wait()
                cols = jnp.full((VEC,), 0, jnp.int32) + (idv - sl)
                for q in range(EMBED // VEC):
                    vals = plsc.load_gather(slab_v.at[slot],
                                            [rows0 + q * VEC, cols])
                    stage_v.at[j, pl.ds(q * VEC, VEC)][...] = vals

            @pl.when(tail)
            def _():
                cols = jnp.full((VEC,), 0, jnp.int32) + (idv - TAIL)
                for q in range(EMBED // VEC):
                    vals = plsc.load_gather(tail_v,
                                            [rows0 + q * VEC, cols])
                    stage_v.at[j, pl.ds(q * VEC, VEC)][...] = vals

        def run_table(t_hbm, tail_v, idx_v, o_hbm):
            @pl.loop(0, per_w // VEC)
            def _grp(g):
                vg = idx_v[pl.ds(g * VEC, VEC)]
                for k in range(VEC + NBUF):
                    if k >= NBUF:
                        extract(t_hbm, tail_v, vg[k - NBUF],
                                g * VEC + (k - NBUF), (k - NBUF) % NBUF)
                    if k < VEC:
                        fetch(t_hbm, vg[k], k % NBUF)

            pltpu.sync_copy(stage_v, o_hbm.at[pl.ds(base, per_w)])

        run_table(tu_hbm, tu_tail_v, ui_v, ou_hbm)
        run_table(ti_hbm, ti_tail_v, ii_v, oi_hbm)

    return gather_kernel(Tu, Ti, Tu_tail, Ti_tail, uids, iids, iota16)


def _mlp_body(u_ref, v_ref, w1_ref, b1_ref, w2_ref, b2_ref, o_ref):
    u = u_ref[...]
    v = v_ref[...]
    x = jnp.concatenate([u, v], axis=1)
    h = jnp.maximum(
        lax.dot_general(x, w1_ref[...], (((1,), (0,)), ((), ())),
                        preferred_element_type=jnp.float32)
        + b1_ref[...], 0.0)
    mlp = jnp.sum(h * w2_ref[...].T, axis=1, keepdims=True)
    dot = jnp.sum(u * v, axis=1, keepdims=True)
    o_ref[...] = dot + mlp + b2_ref[...]


def _tc_mlp(gu, gi, W1, b1, W2, b2):
    B = gu.shape[0]
    BLK = 2048
    grid = (B // BLK,)
    b1r = b1.reshape(1, HIDDEN)
    b2r = b2.reshape(1, 1)
    return pl.pallas_call(
        _mlp_body,
        grid=grid,
        in_specs=[
            pl.BlockSpec((BLK, EMBED), lambda i: (i, 0)),
            pl.BlockSpec((BLK, EMBED), lambda i: (i, 0)),
            pl.BlockSpec((2 * EMBED, HIDDEN), lambda i: (0, 0)),
            pl.BlockSpec((1, HIDDEN), lambda i: (0, 0)),
            pl.BlockSpec((HIDDEN, 1), lambda i: (0, 0)),
            pl.BlockSpec((1, 1), lambda i: (0, 0)),
        ],
        out_specs=pl.BlockSpec((BLK, 1), lambda i: (i, 0)),
        out_shape=jax.ShapeDtypeStruct((B, 1), jnp.float32),
    )(gu, gi, W1, b1r, W2, b2r)


@jax.jit
def kernel(user_ids, item_ids, user_table, item_table, W1, b1, W2, b2):
    Tu = user_table.T
    Ti = item_table.T
    Tu_tail = Tu[:, TAIL:]
    Ti_tail = Ti[:, TAIL:]
    iota16 = jnp.arange(VEC, dtype=jnp.int32)
    gu, gi = _sc_native_gather(Tu, Ti, Tu_tail, Ti_tail,
                               user_ids, item_ids, iota16)
    return _tc_mlp(gu, gi, W1, b1, W2, b2)
